# baseline (speedup 1.0000x reference)
import functools
import jax
import jax.numpy as jnp
from jax import lax
from jax.experimental import pallas as pl
from jax.experimental.pallas import tpu as pltpu
from jax.experimental.pallas import tpu_sc as plsc

V = 8192
D = 1024
B = 4 * 8192
NC = 2
NS = 16
NW = NC * NS
BPW = B // NW
K = 32
NCHUNK = BPW // K
S = 8


NBUF = 3


def _sc_body(idx_hbm, table_hbm, out_hbm, idx_v,
             r0, r1, r2, spmem, g0, g1, g2, w0, w1):
    wid = lax.axis_index("s") * NC + lax.axis_index("c")
    pltpu.sync_copy(idx_hbm.at[wid], idx_v)

    bufs = (r0, r1, r2)
    gsems = (g0, g1, g2)
    wsems = (w0, w1)

    sid = lax.axis_index("s")
    nsub = K // S
    nq = NCHUNK * nsub
    gathers = [None] * NCHUNK
    writes = [None] * nq
    for j in range(min(NBUF, NCHUNK)):
        gathers[j] = pltpu.async_copy(
            table_hbm.at[idx_v.at[j]], bufs[j % NBUF], gsems[j % NBUF])
    for j in range(NCHUNK):
        b = j % NBUF
        gathers[j].wait()
        for t in range(nsub):
            q = j * nsub + t
            s = q % 2
            if q >= 2:
                writes[q - 2].wait()
            pltpu.sync_copy(bufs[b].at[pl.ds(t * S, S)], spmem.at[sid, s])
            writes[q] = pltpu.async_copy(
                spmem.at[sid, s], out_hbm.at[wid, j].at[pl.ds(t * S, S)],
                wsems[s])
        nj = j + NBUF
        if nj < NCHUNK:
            gathers[nj] = pltpu.async_copy(
                table_hbm.at[idx_v.at[nj]], bufs[b], gsems[b])
    for q in range(max(0, nq - 2), nq):
        writes[q].wait()


@jax.jit
def _sc_gather(idx3, table):
    mesh = plsc.VectorSubcoreMesh(core_axis_name="c", subcore_axis_name="s")
    run = pl.kernel(
        _sc_body,
        mesh=mesh,
        out_type=jax.ShapeDtypeStruct((NW, NCHUNK, K, D), jnp.float32),
        scratch_types=(
            [pltpu.VMEM((NCHUNK, K), jnp.int32)]
            + [pltpu.VMEM((K, D), jnp.float32)] * NBUF
            + [pltpu.VMEM_SHARED((NS, 2, S, D), jnp.float32)]
            + [pltpu.SemaphoreType.DMA] * 5
        ),
    )
    return run(idx3, table)


def kernel(X, pos_embed_weight):
    idx3 = X.reshape(NW, NCHUNK, K).astype(jnp.int32)
    out = _sc_gather(idx3, pos_embed_weight)
    return out.reshape(X.shape + (D,))


# speedup vs baseline: 1.0158x; 1.0158x over previous
import functools
import jax
import jax.numpy as jnp
from jax import lax
from jax.experimental import pallas as pl
from jax.experimental.pallas import tpu as pltpu
from jax.experimental.pallas import tpu_sc as plsc

V = 8192
D = 1024
B = 4 * 8192
NC = 2
NS = 16
NW = NC * NS
BPW = B // NW
K = 16
NCHUNK = BPW // K
S = 8
NSUB = K // S
NBUF = 6


def _sc_body(idx_hbm, table_hbm, out_hbm, idx_v, bufs, spmem,
             gsems, dsems, ssems):
    cid = lax.axis_index("c")
    sid = lax.axis_index("s")
    wid = sid * NC + cid
    pltpu.sync_copy(idx_hbm.at[wid], idx_v)

    gathers = [None] * NCHUNK
    dwrites = [None] * NCHUNK
    swrites = []
    for j in range(NBUF):
        gathers[j] = pltpu.async_copy(
            table_hbm.at[idx_v.at[j]], bufs.at[j], gsems.at[j])
    for j in range(NCHUNK):
        b = j % NBUF
        gathers[j].wait()
        if j % 2 == 0:
            dwrites[j] = pltpu.async_copy(
                bufs.at[b], out_hbm.at[wid, j], dsems.at[(j // 2) % 2])
        else:
            for t in range(NSUB):
                q = len(swrites)
                s = q % 2
                if q >= 2:
                    swrites[q - 2].wait()
                pltpu.sync_copy(bufs.at[b].at[pl.ds(t * S, S)],
                                spmem.at[sid, s])
                swrites.append(pltpu.async_copy(
                    spmem.at[sid, s],
                    out_hbm.at[wid, j].at[pl.ds(t * S, S)],
                    ssems.at[s]))
        jj = j - 2
        nj = jj + NBUF
        if jj >= 0 and nj < NCHUNK:
            if jj % 2 == 0:
                dwrites[jj].wait()
            nb = nj % NBUF
            gathers[nj] = pltpu.async_copy(
                table_hbm.at[idx_v.at[nj]], bufs.at[nb], gsems.at[nb])
    for j in range(NCHUNK - 6, NCHUNK, 2):
        dwrites[j].wait()
    for q in range(len(swrites) - 2, len(swrites)):
        swrites[q].wait()


@jax.jit
def _sc_gather(idx3, table):
    mesh = plsc.VectorSubcoreMesh(core_axis_name="c", subcore_axis_name="s")
    run = pl.kernel(
        _sc_body,
        mesh=mesh,
        out_type=jax.ShapeDtypeStruct((NW, NCHUNK, K, D), jnp.float32),
        scratch_types=[
            pltpu.VMEM((NCHUNK, K), jnp.int32),
            pltpu.VMEM((NBUF, K, D), jnp.float32),
            pltpu.VMEM_SHARED((NS, 2, S, D), jnp.float32),
            pltpu.SemaphoreType.DMA((NBUF,)),
            pltpu.SemaphoreType.DMA((2,)),
            pltpu.SemaphoreType.DMA((2,)),
        ],
    )
    return run(idx3, table)


def kernel(X, pos_embed_weight):
    idx3 = X.reshape(NW, NCHUNK, K).astype(jnp.int32)
    out = _sc_gather(idx3, pos_embed_weight)
    return out.reshape(X.shape + (D,))
